# baseline (speedup 1.0000x reference)
import jax
import jax.numpy as jnp
from jax.experimental import pallas as pl
from jax.experimental.pallas import tpu as pltpu

_NUM_VIEWS = 8
_PROMPT_LEN = 50
_DIM = 768
_BATCH = 1024
_BB = 64


def _tc_gather(view_id, prompts):
    def body(idx_ref, tbl_ref, out_ref):
        i = pl.program_id(0)
        for r in range(_BB):
            v = idx_ref[i * _BB + r]
            out_ref[r] = tbl_ref[v]

    return pl.pallas_call(
        body,
        grid_spec=pltpu.PrefetchScalarGridSpec(
            num_scalar_prefetch=1,
            grid=(_BATCH // _BB,),
            in_specs=[
                pl.BlockSpec((_NUM_VIEWS, _PROMPT_LEN, _DIM), lambda i, idx: (0, 0, 0)),
            ],
            out_specs=pl.BlockSpec((_BB, _PROMPT_LEN, _DIM), lambda i, idx: (i, 0, 0)),
        ),
        out_shape=jax.ShapeDtypeStruct((_BATCH, _PROMPT_LEN, _DIM), jnp.float32),
    )(view_id, prompts)


def kernel(view_id, prompts):
    return _tc_gather(view_id.astype(jnp.int32), prompts)
